# baseline (speedup 1.0000x reference)
import jax
import jax.numpy as jnp
from jax.experimental import pallas as pl
from jax.experimental.pallas import tpu as pltpu

_N_PATCHES = (2, 5, 10, 17, 26, 37, 65, 101, 170, 257, 1025)
_CB_CHUNK = 1024
_HI = jax.lax.Precision.HIGHEST


def _dot_hi(a, b):
    return jax.lax.dot_general(a, b, (((1,), (0,)), ((), ())),
                               precision=_HI,
                               preferred_element_type=jnp.float32)


def _dot_bf(a_bf, b_bf, contract_b=0):
    return jax.lax.dot_general(a_bf, b_bf,
                               (((1,), (contract_b,)), ((), ())),
                               preferred_element_type=jnp.float32)


def _nearest_oh(n_out, n_in, n_pad=None):
    rows = n_out if n_pad is None else n_pad
    r2 = jax.lax.broadcasted_iota(jnp.int32, (rows, n_in), 0)
    c2 = jax.lax.broadcasted_iota(jnp.int32, (rows, n_in), 1)
    tgt = jnp.floor((r2.astype(jnp.float32) + 0.5) * float(n_in)
                    / float(n_out))
    tgt = jnp.where(r2 < n_out, tgt, 0.0)
    return (c2.astype(jnp.float32) == tgt).astype(jnp.float32)


def _match(fn, cbn_ref, K, chunk=_CB_CHUNK):
    R = fn.shape[0]
    fn_bf = fn.astype(jnp.bfloat16)
    best_val = jnp.full((R, 1), -jnp.inf, jnp.float32)
    best_idx = jnp.zeros((R, 1), jnp.int32)

    def step(i, carry):
        best_val, best_idx = carry
        c0 = i * chunk
        s = _dot_bf(fn_bf, cbn_ref[pl.ds(c0, chunk), :], contract_b=1)
        m = jnp.max(s, axis=1, keepdims=True)
        ki = jax.lax.broadcasted_iota(jnp.int32, (R, chunk), 1)
        li = jnp.min(jnp.where(s == m, ki, chunk),
                     axis=1, keepdims=True) + c0
        upd = m > best_val
        return (jnp.maximum(best_val, m),
                jnp.where(upd, li, best_idx))

    best_val, best_idx = jax.lax.fori_loop(
        0, K // chunk, step, (best_val, best_idx))
    return best_idx


def _select_part(best_idx, part_ref, K, C, chunk):
    R = best_idx.shape[0]

    def step(i, h):
        c0 = i * chunk
        ki = jax.lax.broadcasted_iota(jnp.int32, (R, chunk), 1) + c0
        oh = (ki == best_idx).astype(jnp.bfloat16)
        return h + _dot_bf(oh, part_ref[pl.ds(c0, chunk), :])

    return jax.lax.fori_loop(0, K // chunk, step,
                             jnp.zeros((R, C), jnp.float32))


def _lookup(best_idx, hi_ref, mid_ref, lo_ref, K, C, chunk=_CB_CHUNK):
    h_hi = _select_part(best_idx, hi_ref, K, C, chunk)
    h_mid = _select_part(best_idx, mid_ref, K, C, chunk)
    h_lo = _select_part(best_idx, lo_ref, K, C, chunk)
    return (h_hi + h_mid) + h_lo


def kernel(f, codebook):
    B, N, C = f.shape
    K, _ = codebook.shape
    scales = [p - 1 for p in _N_PATCHES]
    small = [pn for pn in scales if pn != N]

    cbn_bf = (codebook / (jnp.sum(jnp.square(codebook), axis=-1,
                                  keepdims=True) + 1e-06)
              ).astype(jnp.bfloat16)
    cb_hi = codebook.astype(jnp.bfloat16)
    res1 = codebook - cb_hi.astype(jnp.float32)
    cb_mid = res1.astype(jnp.bfloat16)
    cb_lo = (res1 - cb_mid.astype(jnp.float32)).astype(jnp.bfloat16)

    def body_a(f_ref, hi_ref, mid_ref, lo_ref, cbn_ref, rest_ref,
               hat_ref):
        rest_ref[...] = f_ref[...]
        hat_ref[...] = jnp.zeros((B, N, C), jnp.float32)
        for pn in small:
            pn8 = -(-pn // 8) * 8
            d_oh = _nearest_oh(pn, N, n_pad=pn8)
            u_oh = _nearest_oh(N, pn)

            def up_update(b, h_b):
                h_up = _dot_hi(u_oh, h_b)
                hat_ref[b] = hat_ref[b] + h_up
                rest_ref[b] = rest_ref[b] - h_up

            if pn <= 100:
                f_res = jnp.concatenate(
                    [_dot_hi(d_oh, rest_ref[b]) for b in range(B)], axis=0)
                fn = f_res / (jnp.sum(f_res * f_res, axis=1,
                                      keepdims=True) + 1e-6)
                best_idx = _match(fn, cbn_ref, K, chunk=512)
                h = _lookup(best_idx, hi_ref, mid_ref, lo_ref, K, C,
                            chunk=512)
                for b in range(B):
                    up_update(b, h[b * pn8:b * pn8 + pn, :])
            else:
                for b in range(B):
                    f_res = _dot_hi(d_oh, rest_ref[b])
                    fn = f_res / (jnp.sum(f_res * f_res, axis=1,
                                          keepdims=True) + 1e-6)
                    best_idx = _match(fn, cbn_ref, K)
                    h = _lookup(best_idx, hi_ref, mid_ref, lo_ref, K, C)
                    up_update(b, h[:pn, :])

    def body_b(rest_ref, hat_ref, hi_ref, mid_ref, lo_ref, cbn_ref,
               out_ref):
        f_res = rest_ref[0]
        fn = f_res / (jnp.sum(f_res * f_res, axis=1, keepdims=True) + 1e-6)
        best_idx = _match(fn, cbn_ref, K)
        h = _lookup(best_idx, hi_ref, mid_ref, lo_ref, K, C)
        out_ref[0] = hat_ref[0] + h

    rest8, hat8 = pl.pallas_call(
        body_a,
        grid=(1,),
        in_specs=[
            pl.BlockSpec((B, N, C), lambda i: (0, 0, 0)),
            pl.BlockSpec((K, C), lambda i: (0, 0)),
            pl.BlockSpec((K, C), lambda i: (0, 0)),
            pl.BlockSpec((K, C), lambda i: (0, 0)),
            pl.BlockSpec((K, C), lambda i: (0, 0)),
        ],
        out_specs=[
            pl.BlockSpec((B, N, C), lambda i: (0, 0, 0)),
            pl.BlockSpec((B, N, C), lambda i: (0, 0, 0)),
        ],
        out_shape=[
            jax.ShapeDtypeStruct((B, N, C), jnp.float32),
            jax.ShapeDtypeStruct((B, N, C), jnp.float32),
        ],
    )(f, cb_hi, cb_mid, cb_lo, cbn_bf)

    return pl.pallas_call(
        body_b,
        grid=(B,),
        in_specs=[
            pl.BlockSpec((1, N, C), lambda b: (b, 0, 0)),
            pl.BlockSpec((1, N, C), lambda b: (b, 0, 0)),
            pl.BlockSpec((K, C), lambda b: (0, 0)),
            pl.BlockSpec((K, C), lambda b: (0, 0)),
            pl.BlockSpec((K, C), lambda b: (0, 0)),
            pl.BlockSpec((K, C), lambda b: (0, 0)),
        ],
        out_specs=pl.BlockSpec((1, N, C), lambda b: (b, 0, 0)),
        out_shape=jax.ShapeDtypeStruct((B, N, C), jnp.float32),
    )(rest8, hat8, cb_hi, cb_mid, cb_lo, cbn_bf)


# speedup vs baseline: 1.0322x; 1.0322x over previous
import jax
import jax.numpy as jnp
from jax.experimental import pallas as pl
from jax.experimental.pallas import tpu as pltpu

_N_PATCHES = (2, 5, 10, 17, 26, 37, 65, 101, 170, 257, 1025)
_CB_CHUNK = 1024
_HI = jax.lax.Precision.HIGHEST


def _dot_hi(a, b):
    return jax.lax.dot_general(a, b, (((1,), (0,)), ((), ())),
                               precision=_HI,
                               preferred_element_type=jnp.float32)


def _dot_bf(a_bf, b_bf, contract_b=0):
    return jax.lax.dot_general(a_bf, b_bf,
                               (((1,), (contract_b,)), ((), ())),
                               preferred_element_type=jnp.float32)


def _nearest_oh(n_out, n_in, n_pad=None):
    rows = n_out if n_pad is None else n_pad
    r2 = jax.lax.broadcasted_iota(jnp.int32, (rows, n_in), 0)
    c2 = jax.lax.broadcasted_iota(jnp.int32, (rows, n_in), 1)
    tgt = jnp.floor((r2.astype(jnp.float32) + 0.5) * float(n_in)
                    / float(n_out))
    tgt = jnp.where(r2 < n_out, tgt, 0.0)
    return (c2.astype(jnp.float32) == tgt).astype(jnp.float32)


def _match(fn, cbn_ref, K, chunk=_CB_CHUNK):
    R = fn.shape[0]
    fn_bf = fn.astype(jnp.bfloat16)
    best_val = jnp.full((R, 1), -jnp.inf, jnp.float32)
    best_idx = jnp.zeros((R, 1), jnp.int32)

    def step(i, carry):
        best_val, best_idx = carry
        c0 = i * chunk
        s = _dot_bf(fn_bf, cbn_ref[pl.ds(c0, chunk), :], contract_b=1)
        m = jnp.max(s, axis=1, keepdims=True)
        ki = jax.lax.broadcasted_iota(jnp.int32, (R, chunk), 1)
        li = jnp.min(jnp.where(s == m, ki, chunk),
                     axis=1, keepdims=True) + c0
        upd = m > best_val
        return (jnp.maximum(best_val, m),
                jnp.where(upd, li, best_idx))

    best_val, best_idx = jax.lax.fori_loop(
        0, K // chunk, step, (best_val, best_idx))
    return best_idx


def _lookup(best_idx, cb_ref, K, C, chunk=_CB_CHUNK):
    R = best_idx.shape[0]

    def step(i, h):
        c0 = i * chunk
        ki = jax.lax.broadcasted_iota(jnp.int32, (R, chunk), 1) + c0
        oh = (ki == best_idx).astype(jnp.float32)
        return h + _dot_hi(oh, cb_ref[pl.ds(c0, chunk), :])

    return jax.lax.fori_loop(0, K // chunk, step,
                             jnp.zeros((R, C), jnp.float32))


def kernel(f, codebook):
    B, N, C = f.shape
    K, _ = codebook.shape
    scales = [p - 1 for p in _N_PATCHES]
    small = [pn for pn in scales if pn != N]

    cbn_bf = (codebook / (jnp.sum(jnp.square(codebook), axis=-1,
                                  keepdims=True) + 1e-06)
              ).astype(jnp.bfloat16)
    def body_a(f_ref, cb_ref, cbn_ref, rest_ref, hat_ref):
        rest_ref[...] = f_ref[...]
        hat_ref[...] = jnp.zeros((B, N, C), jnp.float32)
        for pn in small:
            pn8 = -(-pn // 8) * 8
            d_oh = _nearest_oh(pn, N, n_pad=pn8)
            u_oh = _nearest_oh(N, pn)

            def up_update(b, h_b):
                h_up = _dot_hi(u_oh, h_b)
                hat_ref[b] = hat_ref[b] + h_up
                rest_ref[b] = rest_ref[b] - h_up

            if pn <= 100:
                f_res = jnp.concatenate(
                    [_dot_hi(d_oh, rest_ref[b]) for b in range(B)], axis=0)
                fn = f_res / (jnp.sum(f_res * f_res, axis=1,
                                      keepdims=True) + 1e-6)
                best_idx = _match(fn, cbn_ref, K, chunk=512)
                h = _lookup(best_idx, cb_ref, K, C, chunk=512)
                for b in range(B):
                    up_update(b, h[b * pn8:b * pn8 + pn, :])
            else:
                for b in range(B):
                    f_res = _dot_hi(d_oh, rest_ref[b])
                    fn = f_res / (jnp.sum(f_res * f_res, axis=1,
                                          keepdims=True) + 1e-6)
                    best_idx = _match(fn, cbn_ref, K)
                    h = _lookup(best_idx, cb_ref, K, C)
                    up_update(b, h[:pn, :])

    def body_b(rest_ref, hat_ref, cb_ref, cbn_ref, out_ref):
        f_res = rest_ref[0]
        fn = f_res / (jnp.sum(f_res * f_res, axis=1, keepdims=True) + 1e-6)
        best_idx = _match(fn, cbn_ref, K, chunk=2048)
        h = _lookup(best_idx, cb_ref, K, C, chunk=2048)
        out_ref[0] = hat_ref[0] + h

    rest8, hat8 = pl.pallas_call(
        body_a,
        grid=(1,),
        in_specs=[
            pl.BlockSpec((B, N, C), lambda i: (0, 0, 0)),
            pl.BlockSpec((K, C), lambda i: (0, 0)),
            pl.BlockSpec((K, C), lambda i: (0, 0)),
        ],
        out_specs=[
            pl.BlockSpec((B, N, C), lambda i: (0, 0, 0)),
            pl.BlockSpec((B, N, C), lambda i: (0, 0, 0)),
        ],
        out_shape=[
            jax.ShapeDtypeStruct((B, N, C), jnp.float32),
            jax.ShapeDtypeStruct((B, N, C), jnp.float32),
        ],
    )(f, codebook, cbn_bf)

    return pl.pallas_call(
        body_b,
        grid=(B,),
        in_specs=[
            pl.BlockSpec((1, N, C), lambda b: (b, 0, 0)),
            pl.BlockSpec((1, N, C), lambda b: (b, 0, 0)),
            pl.BlockSpec((K, C), lambda b: (0, 0)),
            pl.BlockSpec((K, C), lambda b: (0, 0)),
        ],
        out_specs=pl.BlockSpec((1, N, C), lambda b: (b, 0, 0)),
        out_shape=jax.ShapeDtypeStruct((B, N, C), jnp.float32),
    )(rest8, hat8, codebook, cbn_bf)


# speedup vs baseline: 1.1333x; 1.0979x over previous
import jax
import jax.numpy as jnp
from jax.experimental import pallas as pl
from jax.experimental.pallas import tpu as pltpu

_N_PATCHES = (2, 5, 10, 17, 26, 37, 65, 101, 170, 257, 1025)
_CB_CHUNK = 1024
_HI = jax.lax.Precision.HIGHEST


def _dot_hi(a, b):
    return jax.lax.dot_general(a, b, (((1,), (0,)), ((), ())),
                               precision=_HI,
                               preferred_element_type=jnp.float32)


def _dot_bf(a_bf, b_bf, contract_b=0):
    return jax.lax.dot_general(a_bf, b_bf,
                               (((1,), (contract_b,)), ((), ())),
                               preferred_element_type=jnp.float32)


def _nearest_oh(n_out, n_in, n_pad=None):
    rows = n_out if n_pad is None else n_pad
    r2 = jax.lax.broadcasted_iota(jnp.int32, (rows, n_in), 0)
    c2 = jax.lax.broadcasted_iota(jnp.int32, (rows, n_in), 1)
    tgt = jnp.floor((r2.astype(jnp.float32) + 0.5) * float(n_in)
                    / float(n_out))
    tgt = jnp.where(r2 < n_out, tgt, 0.0)
    return (c2.astype(jnp.float32) == tgt).astype(jnp.float32)


def _match(fn, cbn_ref, K, chunk=_CB_CHUNK):
    R = fn.shape[0]
    fn_bf = fn.astype(jnp.bfloat16)
    best_val = jnp.full((R, 1), -jnp.inf, jnp.float32)
    best_idx = jnp.zeros((R, 1), jnp.int32)

    def step(i, carry):
        best_val, best_idx = carry
        c0 = i * chunk
        s = _dot_bf(fn_bf, cbn_ref[pl.ds(c0, chunk), :], contract_b=1)
        m = jnp.max(s, axis=1, keepdims=True)
        ki = jax.lax.broadcasted_iota(jnp.int32, (R, chunk), 1)
        li = jnp.min(jnp.where(s == m, ki, chunk),
                     axis=1, keepdims=True) + c0
        upd = m > best_val
        return (jnp.maximum(best_val, m),
                jnp.where(upd, li, best_idx))

    best_val, best_idx = jax.lax.fori_loop(
        0, K // chunk, step, (best_val, best_idx))
    return best_idx


def _lookup(best_idx, cb_ref, K, C, chunk=_CB_CHUNK):
    R = best_idx.shape[0]

    def step(i, h):
        c0 = i * chunk
        ki = jax.lax.broadcasted_iota(jnp.int32, (R, chunk), 1) + c0
        oh = (ki == best_idx).astype(jnp.float32)
        return h + _dot_hi(oh, cb_ref[pl.ds(c0, chunk), :])

    return jax.lax.fori_loop(0, K // chunk, step,
                             jnp.zeros((R, C), jnp.float32))


def kernel(f, codebook):
    B, N, C = f.shape
    K, _ = codebook.shape
    scales = [p - 1 for p in _N_PATCHES]
    small = [pn for pn in scales if pn != N]

    cbn_bf = (codebook / (jnp.sum(jnp.square(codebook), axis=-1,
                                  keepdims=True) + 1e-06)
              ).astype(jnp.bfloat16)
    def body_a(f_ref, cb_ref, cbn_ref, rest_ref, hat_ref):
        rest_ref[...] = f_ref[...]
        hat_ref[...] = jnp.zeros((B, N, C), jnp.float32)
        for pn in small:
            pn8 = -(-pn // 8) * 8
            d_oh = _nearest_oh(pn, N, n_pad=pn8)
            u_oh = _nearest_oh(N, pn)

            def up_update(b, h_b):
                h_up = _dot_hi(u_oh, h_b)
                hat_ref[b] = hat_ref[b] + h_up
                rest_ref[b] = rest_ref[b] - h_up

            if pn <= 100:
                f_res = jnp.concatenate(
                    [_dot_hi(d_oh, rest_ref[b]) for b in range(B)], axis=0)
                fn = f_res / (jnp.sum(f_res * f_res, axis=1,
                                      keepdims=True) + 1e-6)
                best_idx = _match(fn, cbn_ref, K, chunk=1024)
                h = _lookup(best_idx, cb_ref, K, C, chunk=1024)
                for b in range(B):
                    up_update(b, h[b * pn8:b * pn8 + pn, :])
            else:
                for b in range(B):
                    f_res = _dot_hi(d_oh, rest_ref[b])
                    fn = f_res / (jnp.sum(f_res * f_res, axis=1,
                                          keepdims=True) + 1e-6)
                    best_idx = _match(fn, cbn_ref, K, chunk=2048)
                    h = _lookup(best_idx, cb_ref, K, C, chunk=2048)
                    up_update(b, h[:pn, :])

    def body_b(rest_ref, hat_ref, cb_ref, cbn_ref, out_ref):
        f_res = rest_ref[0]
        fn = f_res / (jnp.sum(f_res * f_res, axis=1, keepdims=True) + 1e-6)
        best_idx = _match(fn, cbn_ref, K, chunk=2048)
        h = _lookup(best_idx, cb_ref, K, C, chunk=2048)
        out_ref[0] = hat_ref[0] + h

    rest8, hat8 = pl.pallas_call(
        body_a,
        grid=(1,),
        in_specs=[
            pl.BlockSpec((B, N, C), lambda i: (0, 0, 0)),
            pl.BlockSpec((K, C), lambda i: (0, 0)),
            pl.BlockSpec((K, C), lambda i: (0, 0)),
        ],
        out_specs=[
            pl.BlockSpec((B, N, C), lambda i: (0, 0, 0)),
            pl.BlockSpec((B, N, C), lambda i: (0, 0, 0)),
        ],
        out_shape=[
            jax.ShapeDtypeStruct((B, N, C), jnp.float32),
            jax.ShapeDtypeStruct((B, N, C), jnp.float32),
        ],
    )(f, codebook, cbn_bf)

    return pl.pallas_call(
        body_b,
        grid=(B,),
        in_specs=[
            pl.BlockSpec((1, N, C), lambda b: (b, 0, 0)),
            pl.BlockSpec((1, N, C), lambda b: (b, 0, 0)),
            pl.BlockSpec((K, C), lambda b: (0, 0)),
            pl.BlockSpec((K, C), lambda b: (0, 0)),
        ],
        out_specs=pl.BlockSpec((1, N, C), lambda b: (b, 0, 0)),
        out_shape=jax.ShapeDtypeStruct((B, N, C), jnp.float32),
    )(rest8, hat8, codebook, cbn_bf)


# speedup vs baseline: 1.1955x; 1.0548x over previous
import jax
import jax.numpy as jnp
from jax.experimental import pallas as pl
from jax.experimental.pallas import tpu as pltpu

_N_PATCHES = (2, 5, 10, 17, 26, 37, 65, 101, 170, 257, 1025)
_CB_CHUNK = 1024
_HI = jax.lax.Precision.HIGHEST


def _dot_hi(a, b):
    return jax.lax.dot_general(a, b, (((1,), (0,)), ((), ())),
                               precision=_HI,
                               preferred_element_type=jnp.float32)


def _dot_bf(a_bf, b_bf, contract_b=0):
    return jax.lax.dot_general(a_bf, b_bf,
                               (((1,), (contract_b,)), ((), ())),
                               preferred_element_type=jnp.float32)


def _nearest_oh(n_out, n_in, n_pad=None):
    rows = n_out if n_pad is None else n_pad
    r2 = jax.lax.broadcasted_iota(jnp.int32, (rows, n_in), 0)
    c2 = jax.lax.broadcasted_iota(jnp.int32, (rows, n_in), 1)
    tgt = jnp.floor((r2.astype(jnp.float32) + 0.5) * float(n_in)
                    / float(n_out))
    tgt = jnp.where(r2 < n_out, tgt, 0.0)
    return (c2.astype(jnp.float32) == tgt).astype(jnp.float32)


def _match(fn, cbn_ref, K, chunk=_CB_CHUNK):
    R = fn.shape[0]
    fn_bf = fn.astype(jnp.bfloat16)
    best_val = jnp.full((R, 1), -jnp.inf, jnp.float32)
    best_idx = jnp.zeros((R, 1), jnp.int32)

    def step(i, carry):
        best_val, best_idx = carry
        c0 = i * chunk
        s = _dot_bf(fn_bf, cbn_ref[pl.ds(c0, chunk), :], contract_b=1)
        m = jnp.max(s, axis=1, keepdims=True)
        ki = jax.lax.broadcasted_iota(jnp.int32, (R, chunk), 1)
        li = jnp.min(jnp.where(s == m, ki, chunk),
                     axis=1, keepdims=True) + c0
        upd = m > best_val
        return (jnp.maximum(best_val, m),
                jnp.where(upd, li, best_idx))

    best_val, best_idx = jax.lax.fori_loop(
        0, K // chunk, step, (best_val, best_idx))
    return best_idx


def _lookup(best_idx, cb_ref, K, C, chunk=_CB_CHUNK):
    R = best_idx.shape[0]

    def step(i, h):
        c0 = i * chunk
        ki = jax.lax.broadcasted_iota(jnp.int32, (R, chunk), 1) + c0
        oh = (ki == best_idx).astype(jnp.float32)
        return h + _dot_hi(oh, cb_ref[pl.ds(c0, chunk), :])

    return jax.lax.fori_loop(0, K // chunk, step,
                             jnp.zeros((R, C), jnp.float32))


def kernel(f, codebook):
    B, N, C = f.shape
    K, _ = codebook.shape
    scales = [p - 1 for p in _N_PATCHES]
    small = [pn for pn in scales if pn != N]

    cbn_bf = (codebook / (jnp.sum(jnp.square(codebook), axis=-1,
                                  keepdims=True) + 1e-06)
              ).astype(jnp.bfloat16)
    def body_a(f_ref, cb_ref, cbn_ref, rest_ref, hat_ref):
        rest_ref[...] = f_ref[...]
        hat_ref[...] = jnp.zeros((B, N, C), jnp.float32)
        for pn in small:
            pn8 = -(-pn // 8) * 8
            d_oh = _nearest_oh(pn, N, n_pad=pn8)
            u_oh = _nearest_oh(N, pn)

            def up_update(b, h_b):
                h_up = _dot_hi(u_oh, h_b)
                hat_ref[b] = hat_ref[b] + h_up
                rest_ref[b] = rest_ref[b] - h_up

            if pn <= 100:
                f_res = jnp.concatenate(
                    [_dot_hi(d_oh, rest_ref[b]) for b in range(B)], axis=0)
                fn = f_res / (jnp.sum(f_res * f_res, axis=1,
                                      keepdims=True) + 1e-6)
                best_idx = _match(fn, cbn_ref, K, chunk=2048)
                h = _lookup(best_idx, cb_ref, K, C, chunk=2048)
                for b in range(B):
                    up_update(b, h[b * pn8:b * pn8 + pn, :])
            else:
                for b in range(B):
                    f_res = _dot_hi(d_oh, rest_ref[b])
                    fn = f_res / (jnp.sum(f_res * f_res, axis=1,
                                          keepdims=True) + 1e-6)
                    best_idx = _match(fn, cbn_ref, K, chunk=4096)
                    h = _lookup(best_idx, cb_ref, K, C, chunk=4096)
                    up_update(b, h[:pn, :])

    def body_b(rest_ref, hat_ref, cb_ref, cbn_ref, out_ref):
        f_res = rest_ref[0]
        fn = f_res / (jnp.sum(f_res * f_res, axis=1, keepdims=True) + 1e-6)
        best_idx = _match(fn, cbn_ref, K, chunk=2048)
        h = _lookup(best_idx, cb_ref, K, C, chunk=2048)
        out_ref[0] = hat_ref[0] + h

    rest8, hat8 = pl.pallas_call(
        body_a,
        grid=(1,),
        in_specs=[
            pl.BlockSpec((B, N, C), lambda i: (0, 0, 0)),
            pl.BlockSpec((K, C), lambda i: (0, 0)),
            pl.BlockSpec((K, C), lambda i: (0, 0)),
        ],
        out_specs=[
            pl.BlockSpec((B, N, C), lambda i: (0, 0, 0)),
            pl.BlockSpec((B, N, C), lambda i: (0, 0, 0)),
        ],
        out_shape=[
            jax.ShapeDtypeStruct((B, N, C), jnp.float32),
            jax.ShapeDtypeStruct((B, N, C), jnp.float32),
        ],
    )(f, codebook, cbn_bf)

    return pl.pallas_call(
        body_b,
        grid=(B,),
        in_specs=[
            pl.BlockSpec((1, N, C), lambda b: (b, 0, 0)),
            pl.BlockSpec((1, N, C), lambda b: (b, 0, 0)),
            pl.BlockSpec((K, C), lambda b: (0, 0)),
            pl.BlockSpec((K, C), lambda b: (0, 0)),
        ],
        out_specs=pl.BlockSpec((1, N, C), lambda b: (b, 0, 0)),
        out_shape=jax.ShapeDtypeStruct((B, N, C), jnp.float32),
    )(rest8, hat8, codebook, cbn_bf)
